# baseline (speedup 1.0000x reference)
import functools

import numpy as np

import jax
import jax.numpy as jnp
from jax import lax
from jax.experimental import pallas as pl
from jax.experimental.pallas import tpu as pltpu
from jax.experimental.pallas import tpu_sc as plsc

N_NODES = 10000
D = 128
NPAD = 10240
NW = 32
K = 128
STEPS = 82
W = STEPS // 2
NEP = NW * STEPS * K
RPT = NPAD // 16
BLK = 512

_mesh = plsc.VectorSubcoreMesh(core_axis_name="c", subcore_axis_name="s")


@functools.partial(
    pl.kernel,
    out_type=jax.ShapeDtypeStruct((2, NPAD, D), jnp.float32),
    mesh=_mesh,
    scratch_types=[
        pltpu.VMEM_SHARED((NPAD, D), jnp.float32),
        pltpu.VMEM((W, K), jnp.int32),
        pltpu.VMEM((W, K), jnp.int32),
        pltpu.VMEM((K, D), jnp.float32),
        pltpu.VMEM((K, D), jnp.float32),
        pltpu.SemaphoreType.DMA,
        pltpu.SemaphoreType.DMA,
    ],
)
def _gather_scatter_kernel(h_hbm, src_hbm, dst_hbm, out_hbm,
                           acc_sh, src_v, dst_v, msg0, msg1, sem0, sem1):
    c = lax.axis_index("c")
    s = lax.axis_index("s")
    wid = c * 16 + s

    def zero_row(i, carry):
        for t in range(D // 16):
            msg0[i, pl.ds(t * 16, 16)] = jnp.zeros((16,), jnp.float32)
        return carry

    lax.fori_loop(0, K, zero_row, 0)
    for t in range(RPT // K):
        pltpu.sync_copy(msg0, acc_sh.at[pl.ds(s * RPT + t * K, K)])
    plsc.subcore_barrier()

    for p in range(STEPS // W):
        pltpu.sync_copy(src_hbm.at[wid, p], src_v)
        pltpu.sync_copy(dst_hbm.at[wid, p], dst_v)
        pltpu.async_copy(h_hbm.at[src_v.at[0]], msg0, sem0)

        def step(i, carry):
            ja = 2 * i
            jb = 2 * i + 1
            jc = jnp.minimum(2 * i + 2, W - 1)
            pltpu.make_async_copy(h_hbm.at[src_v.at[0]], msg0, sem0).wait()
            pltpu.async_copy(h_hbm.at[src_v.at[jb]], msg1, sem1)
            pltpu.sync_copy(msg0, acc_sh.at[dst_v.at[ja]], add=True)
            pltpu.make_async_copy(h_hbm.at[src_v.at[0]], msg1, sem1).wait()
            pltpu.async_copy(h_hbm.at[src_v.at[jc]], msg0, sem0)
            pltpu.sync_copy(msg1, acc_sh.at[dst_v.at[jb]], add=True)
            return carry

        lax.fori_loop(0, W // 2, step, 0)
        pltpu.make_async_copy(h_hbm.at[src_v.at[0]], msg0, sem0).wait()
        pltpu.sync_copy(msg0, acc_sh.at[dst_v.at[W - 1]], add=True)
    plsc.subcore_barrier()

    for t in range(RPT // K):
        pltpu.sync_copy(acc_sh.at[pl.ds(s * RPT + t * K, K)], msg0)
        pltpu.sync_copy(msg0, out_hbm.at[c, pl.ds(s * RPT + t * K, K), :])


def _t_dis_body(degp_ref, out_ref):
    deg = degp_ref[0, :, 0:1] + degp_ref[1, :, 0:1]
    dis = jnp.where(deg > 0.5, lax.rsqrt(deg), 0.0)
    out_ref[...] = jnp.broadcast_to(dis, out_ref.shape)


def _t_first_body(x_ref, w_ref, dis_ref, out_ref):
    dis = dis_ref[:, 0:1]
    h = jnp.dot(x_ref[...], w_ref[...], preferred_element_type=jnp.float32)
    out_ref[...] = h * dis


def _t_mid_body(p_ref, dis_ref, b_ref, w_ref, out_ref):
    dis = dis_ref[:, 0:1]
    z = jnp.maximum((p_ref[0] + p_ref[1]) * dis + b_ref[...], 0.0)
    out_ref[...] = jnp.dot(z, w_ref[...], preferred_element_type=jnp.float32) * dis


def _t_final_body(p_ref, dis_ref, b_ref, out_ref):
    dis = dis_ref[:, 0:1]
    out_ref[...] = (p_ref[0] + p_ref[1]) * dis + b_ref[...]


_spec_rows = pl.BlockSpec((BLK, D), lambda i: (i, 0))
_spec_dis = pl.BlockSpec((BLK, 16), lambda i: (i, 0))
_spec_w = pl.BlockSpec((D, D), lambda i: (0, 0))
_spec_part = pl.BlockSpec((2, BLK, D), lambda i: (0, i, 0))
_spec_b = pl.BlockSpec((1, D), lambda i: (0, 0))
_out_rows = jax.ShapeDtypeStruct((NPAD, D), jnp.float32)


def _t_dis(degp):
    return pl.pallas_call(
        _t_dis_body, grid=(NPAD // BLK,),
        in_specs=[_spec_part],
        out_specs=_spec_dis,
        out_shape=jax.ShapeDtypeStruct((NPAD, 16), jnp.float32),
    )(degp)


def _t_first(x_pad, w, dis):
    return pl.pallas_call(
        _t_first_body, grid=(NPAD // BLK,),
        in_specs=[_spec_rows, _spec_w, _spec_dis],
        out_specs=_spec_rows, out_shape=_out_rows,
    )(x_pad, w, dis)


def _t_mid(p, dis, b, w):
    return pl.pallas_call(
        _t_mid_body, grid=(NPAD // BLK,),
        in_specs=[_spec_part, _spec_dis, _spec_b, _spec_w],
        out_specs=_spec_rows, out_shape=_out_rows,
    )(p, dis, b, w)


def _t_final(p, dis, b):
    fblk = 1000
    return pl.pallas_call(
        _t_final_body, grid=(N_NODES // fblk,),
        in_specs=[
            pl.BlockSpec((2, fblk, D), lambda i: (0, i, 0)),
            pl.BlockSpec((fblk, 16), lambda i: (i, 0)),
            _spec_b,
        ],
        out_specs=pl.BlockSpec((fblk, D), lambda i: (i, 0)),
        out_shape=jax.ShapeDtypeStruct((N_NODES, D), jnp.float32),
    )(p, dis, b)


_PAD_E = NEP - (N_NODES + 320000)
_TAIL_SRC = np.concatenate([
    np.arange(N_NODES, dtype=np.int32),
    np.arange(_PAD_E, dtype=np.int32) % N_NODES,
])
_TAIL_DST = np.concatenate([
    np.arange(N_NODES, dtype=np.int32),
    N_NODES + np.arange(_PAD_E, dtype=np.int32) % (NPAD - N_NODES),
])
_SRC_SEQ = (np.arange(NEP, dtype=np.int32) % N_NODES).reshape(
    NW, STEPS // W, W, K)
_ONES_TABLE = np.ones((NPAD, D), dtype=np.float32)


def kernel(x, adj_t, W1, b1, W2, b2, W3, b3):
    adj = adj_t.astype(jnp.int32)
    src_t = jnp.concatenate([adj[0], jnp.asarray(_TAIL_SRC)]).reshape(
        NW, STEPS // W, W, K)
    dst_t = jnp.concatenate([adj[1], jnp.asarray(_TAIL_DST)]).reshape(
        NW, STEPS // W, W, K)
    x_pad = jnp.zeros((NPAD, D), jnp.float32).at[:N_NODES].set(x)

    degp = _gather_scatter_kernel(
        jnp.asarray(_ONES_TABLE), jnp.asarray(_SRC_SEQ), dst_t)
    dis = _t_dis(degp)
    h = _t_first(x_pad, W1, dis)
    p = _gather_scatter_kernel(h, src_t, dst_t)
    h = _t_mid(p, dis, b1.reshape(1, D), W2)
    p = _gather_scatter_kernel(h, src_t, dst_t)
    h = _t_mid(p, dis, b2.reshape(1, D), W3)
    p = _gather_scatter_kernel(h, src_t, dst_t)
    return _t_final(p, dis, b3.reshape(1, D))


# speedup vs baseline: 1.0115x; 1.0115x over previous
import functools

import numpy as np

import jax
import jax.numpy as jnp
from jax import lax
from jax.experimental import pallas as pl
from jax.experimental.pallas import tpu as pltpu
from jax.experimental.pallas import tpu_sc as plsc

N_NODES = 10000
D = 128
NPAD = 10240
NW = 32
K = 128
STEPS = 82
W = STEPS // 2
NEP = NW * STEPS * K
RPT = NPAD // 16
BLK = 512

_mesh = plsc.VectorSubcoreMesh(core_axis_name="c", subcore_axis_name="s")


@functools.partial(
    pl.kernel,
    out_type=jax.ShapeDtypeStruct((2, NPAD, D), jnp.float32),
    mesh=_mesh,
    scratch_types=[
        pltpu.VMEM_SHARED((NPAD, D), jnp.float32),
        pltpu.VMEM((W, K), jnp.int32),
        pltpu.VMEM((W, K), jnp.int32),
        pltpu.VMEM((K, D), jnp.float32),
        pltpu.VMEM((K, D), jnp.float32),
        pltpu.SemaphoreType.DMA,
        pltpu.SemaphoreType.DMA,
    ],
)
def _gather_scatter_kernel(h_hbm, src_hbm, dst_hbm, out_hbm,
                           acc_sh, src_v, dst_v, msg0, msg1, sem0, sem1):
    c = lax.axis_index("c")
    s = lax.axis_index("s")
    wid = c * 16 + s

    def zero_row(i, carry):
        for t in range(D // 16):
            msg0[i, pl.ds(t * 16, 16)] = jnp.zeros((16,), jnp.float32)
        return carry

    lax.fori_loop(0, K, zero_row, 0)
    for t in range(RPT // K):
        pltpu.sync_copy(msg0, acc_sh.at[pl.ds(s * RPT + t * K, K)])
    plsc.subcore_barrier()

    for p in range(STEPS // W):
        pltpu.sync_copy(src_hbm.at[wid, p], src_v)
        pltpu.sync_copy(dst_hbm.at[wid, p], dst_v)
        pltpu.async_copy(h_hbm.at[src_v.at[0]], msg0, sem0)

        def step(i, carry):
            ja = 2 * i
            jb = 2 * i + 1
            jc = jnp.minimum(2 * i + 2, W - 1)
            pltpu.make_async_copy(h_hbm.at[src_v.at[0]], msg0, sem0).wait()
            pltpu.async_copy(h_hbm.at[src_v.at[jb]], msg1, sem1)
            pltpu.sync_copy(msg0, acc_sh.at[dst_v.at[ja]], add=True)
            pltpu.make_async_copy(h_hbm.at[src_v.at[0]], msg1, sem1).wait()
            pltpu.async_copy(h_hbm.at[src_v.at[jc]], msg0, sem0)
            pltpu.sync_copy(msg1, acc_sh.at[dst_v.at[jb]], add=True)
            return carry

        lax.fori_loop(0, W // 2, step, 0)
        pltpu.make_async_copy(h_hbm.at[src_v.at[0]], msg0, sem0).wait()
        pltpu.sync_copy(msg0, acc_sh.at[dst_v.at[W - 1]], add=True)
    plsc.subcore_barrier()

    for t in range(RPT // K):
        pltpu.sync_copy(acc_sh.at[pl.ds(s * RPT + t * K, K)], msg0)
        pltpu.sync_copy(msg0, out_hbm.at[c, pl.ds(s * RPT + t * K, K), :])


def _dis_from(degp_ref):
    deg = degp_ref[0, :, 0:1] + degp_ref[1, :, 0:1]
    return jnp.where(deg > 0.5, lax.rsqrt(deg), 0.0)


def _t_first_body(x_ref, w_ref, degp_ref, out_ref):
    dis = _dis_from(degp_ref)
    h = jnp.dot(x_ref[...], w_ref[...], preferred_element_type=jnp.float32)
    out_ref[...] = h * dis


def _t_mid_body(p_ref, degp_ref, b_ref, w_ref, out_ref):
    dis = _dis_from(degp_ref)
    z = jnp.maximum((p_ref[0] + p_ref[1]) * dis + b_ref[...], 0.0)
    out_ref[...] = jnp.dot(z, w_ref[...], preferred_element_type=jnp.float32) * dis


def _t_final_body(p_ref, degp_ref, b_ref, out_ref):
    dis = _dis_from(degp_ref)
    out_ref[...] = (p_ref[0] + p_ref[1]) * dis + b_ref[...]


_spec_rows = pl.BlockSpec((BLK, D), lambda i: (i, 0))
_spec_w = pl.BlockSpec((D, D), lambda i: (0, 0))
_spec_part = pl.BlockSpec((2, BLK, D), lambda i: (0, i, 0))
_spec_b = pl.BlockSpec((1, D), lambda i: (0, 0))
_out_rows = jax.ShapeDtypeStruct((NPAD, D), jnp.float32)


def _t_first(x_pad, w, degp):
    return pl.pallas_call(
        _t_first_body, grid=(NPAD // BLK,),
        in_specs=[_spec_rows, _spec_w, _spec_part],
        out_specs=_spec_rows, out_shape=_out_rows,
    )(x_pad, w, degp)


def _t_mid(p, degp, b, w):
    return pl.pallas_call(
        _t_mid_body, grid=(NPAD // BLK,),
        in_specs=[_spec_part, _spec_part, _spec_b, _spec_w],
        out_specs=_spec_rows, out_shape=_out_rows,
    )(p, degp, b, w)


def _t_final(p, degp, b):
    fblk = 1000
    return pl.pallas_call(
        _t_final_body, grid=(N_NODES // fblk,),
        in_specs=[
            pl.BlockSpec((2, fblk, D), lambda i: (0, i, 0)),
            pl.BlockSpec((2, fblk, D), lambda i: (0, i, 0)),
            _spec_b,
        ],
        out_specs=pl.BlockSpec((fblk, D), lambda i: (i, 0)),
        out_shape=jax.ShapeDtypeStruct((N_NODES, D), jnp.float32),
    )(p, degp, b)


_PAD_E = NEP - (N_NODES + 320000)
_TAIL_SRC = np.concatenate([
    np.arange(N_NODES, dtype=np.int32),
    np.arange(_PAD_E, dtype=np.int32) % N_NODES,
])
_TAIL_DST = np.concatenate([
    np.arange(N_NODES, dtype=np.int32),
    N_NODES + np.arange(_PAD_E, dtype=np.int32) % (NPAD - N_NODES),
])
_SRC_SEQ = (np.arange(NEP, dtype=np.int32) % N_NODES).reshape(
    NW, STEPS // W, W, K)
_ONES_TABLE = np.ones((NPAD, D), dtype=np.float32)


def kernel(x, adj_t, W1, b1, W2, b2, W3, b3):
    adj = adj_t.astype(jnp.int32)
    src_t = jnp.concatenate([adj[0], jnp.asarray(_TAIL_SRC)]).reshape(
        NW, STEPS // W, W, K)
    dst_t = jnp.concatenate([adj[1], jnp.asarray(_TAIL_DST)]).reshape(
        NW, STEPS // W, W, K)
    x_pad = jnp.zeros((NPAD, D), jnp.float32).at[:N_NODES].set(x)

    degp = _gather_scatter_kernel(
        jnp.asarray(_ONES_TABLE), jnp.asarray(_SRC_SEQ), dst_t)
    h = _t_first(x_pad, W1, degp)
    p = _gather_scatter_kernel(h, src_t, dst_t)
    h = _t_mid(p, degp, b1.reshape(1, D), W2)
    p = _gather_scatter_kernel(h, src_t, dst_t)
    h = _t_mid(p, degp, b2.reshape(1, D), W3)
    p = _gather_scatter_kernel(h, src_t, dst_t)
    return _t_final(p, degp, b3.reshape(1, D))
